# baseline (speedup 1.0000x reference)
import functools

import jax
import jax.numpy as jnp
from jax import lax
from jax.experimental import pallas as pl
from jax.experimental.pallas import tpu as pltpu
from jax.experimental.pallas import tpu_sc as plsc

N = 10000
E = 320000
D = 128
G = 256
L = 3

NCORES = 2
NSUB = 16
TILES = NCORES * NSUB
EPT = E // TILES
CH = 125
EPS = E // NSUB
NCH = EPS // CH
WIN = 16
NWIN = NCH // WIN
DH = D // 2
RPT = 632
NP = RPT * NSUB

BLK = 1000
NB = N // BLK

RB = 400
RG = N // RB
NREP = 5
NTASK = RG * NREP
TPW = -(-NTASK // TILES)


def _sc_segsum(hh, src_r, dst_r, zrows):
    mesh = plsc.VectorSubcoreMesh(core_axis_name="c", subcore_axis_name="s")

    nbuf = 2

    @functools.partial(
        pl.kernel,
        out_type=jax.ShapeDtypeStruct((NCORES, NP, DH), jnp.float32),
        mesh=mesh,
        scratch_types=[
            pltpu.VMEM_SHARED((N, DH), jnp.float32),
            pltpu.VMEM_SHARED((NP, DH), jnp.float32),
            pltpu.VMEM((WIN, CH), jnp.int32),
            pltpu.VMEM((WIN, CH), jnp.int32),
            [pltpu.VMEM((CH, DH), jnp.float32)] * nbuf,
            [pltpu.SemaphoreType.DMA] * nbuf,
            [pltpu.SemaphoreType.DMA] * nbuf,
        ],
    )
    def k(hh_hbm, src_hbm, dst_hbm, z_hbm, out_hbm, hspm, acc, sidx, didx,
          rows, gsems, ssems):
        c = lax.axis_index("c")
        s = lax.axis_index("s")
        pltpu.sync_copy(z_hbm, acc.at[pl.ds(s * RPT, RPT)])

        @pl.when(s < 10)
        def _stage():
            pltpu.sync_copy(hh_hbm.at[c, pl.ds(s * 1000, 1000), :],
                            hspm.at[pl.ds(s * 1000, 1000)])

        plsc.subcore_barrier()

        def win(g, carry):
            pltpu.sync_copy(src_hbm.at[s, g], sidx)
            pltpu.sync_copy(dst_hbm.at[s, g], didx)

            def body(i, carry2):
                gd = []
                for b in range(nbuf):
                    gd.append(pltpu.async_copy(
                        hspm.at[sidx.at[i * nbuf + b]], rows[b], gsems[b]))
                sd = []
                for b in range(nbuf):
                    gd[b].wait()
                    sd.append(pltpu.async_copy(
                        rows[b], acc.at[didx.at[i * nbuf + b]], ssems[b],
                        add=True))
                for b in range(nbuf):
                    sd[b].wait()
                return carry2

            lax.fori_loop(0, WIN // nbuf, body, 0)
            return carry

        lax.fori_loop(0, NWIN, win, 0)
        plsc.subcore_barrier()
        pltpu.sync_copy(acc.at[pl.ds(s * RPT, RPT)],
                        out_hbm.at[c, pl.ds(s * RPT, RPT)])

    return k(hh, src_r, dst_r, zrows)


def _sc_segmax(rep, batch, ninf):
    mesh = plsc.VectorSubcoreMesh(core_axis_name="c", subcore_axis_name="s")

    @functools.partial(
        pl.kernel,
        out_type=jax.ShapeDtypeStruct((RG, G, NREP * D), jnp.float32),
        mesh=mesh,
        scratch_types=[
            pltpu.VMEM((RB,), jnp.int32),
            pltpu.VMEM((RB, D), jnp.float32),
            pltpu.VMEM((G, D), jnp.float32),
        ],
        compiler_params=pltpu.CompilerParams(needs_layout_passes=False),
    )
    def k(rep_hbm, batch_hbm, ninf_hbm, out_hbm, ids, buf, acc):
        c = lax.axis_index("c")
        s = lax.axis_index("s")
        w = c * NSUB + s
        lanes = lax.iota(jnp.int32, 16)
        for t in range(TPW):
            task = w + t * TILES

            @pl.when(task < NTASK)
            def _run():
                rg = task // NREP
                a = task % NREP
                pltpu.sync_copy(batch_hbm.at[pl.ds(rg * RB, RB)], ids)
                pltpu.sync_copy(ninf_hbm, acc)
                pltpu.sync_copy(rep_hbm.at[a, pl.ds(rg * RB, RB), :], buf)

                id0 = plsc.load_gather(ids, [jnp.zeros((16,), jnp.int32)])
                init = (id0,) + tuple(
                    jnp.full((16,), -jnp.inf, jnp.float32)
                    for _ in range(D // 16))

                def body(i, carry):
                    pid = carry[0]
                    idv = plsc.load_gather(ids, [jnp.full((16,), i,
                                                          jnp.int32)])
                    new = pid != idv
                    out = [idv]
                    for kk in range(D // 16):
                        rmax = carry[1 + kk]
                        plsc.store_scatter(acc, [pid, lanes + (kk * 16)],
                                           rmax, mask=new)
                        rv = buf[i, pl.ds(kk * 16, 16)]
                        out.append(jnp.where(new, rv, jnp.maximum(rmax, rv)))
                    return tuple(out)

                fin = lax.fori_loop(0, RB, body, init)
                for kk in range(D // 16):
                    plsc.store_scatter(acc, [fin[0], lanes + (kk * 16)],
                                       fin[1 + kk])
                pltpu.sync_copy(acc, out_hbm.at[rg, :, pl.ds(a * D, D)])

    return k(rep, batch, ninf)


def _tc_layer(h, part, m, e, w1t, b1, w2t, b2, wiht, whht, bih, bhh,
              bng, bnb, lng, lnb):
    def body(h_ref, p_ref, m_ref, e_ref, w1_ref, b1_ref, w2_ref, b2_ref,
             wih_ref, whh_ref, bih_ref, bhh_ref, bng_ref, bnb_ref, lng_ref,
             lnb_ref, h_out, m_out, e_out, hh_out, xg_ref, st_ref):
        p = pl.program_id(0)
        i = pl.program_id(1)

        @pl.when(p == 0)
        def _phase_a():
            gin = h_ref[...] + jnp.concatenate([p_ref[0], p_ref[1]], axis=-1)
            t = jnp.maximum(
                jnp.dot(gin, w1_ref[...], preferred_element_type=jnp.float32)
                + b1_ref[...], 0.0)
            xg = jnp.maximum(
                jnp.dot(t, w2_ref[...], preferred_element_type=jnp.float32)
                + b2_ref[...], 0.0)
            xg_ref[pl.ds(i * BLK, BLK), :] = xg

            @pl.when(i == 0)
            def _init():
                st_ref[...] = jnp.zeros_like(st_ref)

            st_ref[0:1, :] = st_ref[0:1, :] + jnp.sum(xg, axis=0,
                                                      keepdims=True)
            st_ref[1:2, :] = st_ref[1:2, :] + jnp.sum(xg * xg, axis=0,
                                                      keepdims=True)

        @pl.when(p == 1)
        def _phase_b():
            mean = st_ref[0:1, :] * (1.0 / N)
            var = st_ref[1:2, :] * (1.0 / N) - mean * mean
            xn = ((xg_ref[pl.ds(i * BLK, BLK), :] - mean)
                  * lax.rsqrt(var + 1e-5) * bng_ref[...] + bnb_ref[...])
            hprev = h_ref[...]
            gi = jnp.dot(xn, wih_ref[...],
                         preferred_element_type=jnp.float32) + bih_ref[...]
            gh = jnp.dot(hprev, whh_ref[...],
                         preferred_element_type=jnp.float32) + bhh_ref[...]
            r = jax.nn.sigmoid(gi[:, :D] + gh[:, :D])
            z = jax.nn.sigmoid(gi[:, D:2 * D] + gh[:, D:2 * D])
            n = jnp.tanh(gi[:, 2 * D:] + r * gh[:, 2 * D:])
            hn = (1.0 - z) * n + z * hprev
            mu = jnp.mean(hn, axis=1, keepdims=True)
            v2 = jnp.mean((hn - mu) * (hn - mu), axis=1, keepdims=True)
            hl = ((hn - mu) * lax.rsqrt(v2 + 1e-5) * lng_ref[...]
                  + lnb_ref[...])
            h_out[...] = hl
            m_out[...] = m_ref[...] * hl
            e_out[...] = e_ref[...] + hl
            hh_out[0] = hl[:, :DH]
            hh_out[1] = hl[:, DH:]

    blk = lambda: pl.BlockSpec((BLK, D), lambda p, i: (i, 0))
    out_blk = lambda: pl.BlockSpec((BLK, D), lambda p, i: (p * i, 0))
    vec = lambda n: pl.BlockSpec((1, n), lambda p, i: (0, 0))
    return pl.pallas_call(
        body,
        grid=(2, NB),
        in_specs=[
            blk(),
            pl.BlockSpec((NCORES, BLK, DH), lambda p, i: (0, (1 - p) * i, 0)),
            blk(), blk(),
            pl.BlockSpec((D, D), lambda p, i: (0, 0)),
            vec(D),
            pl.BlockSpec((D, D), lambda p, i: (0, 0)),
            vec(D),
            pl.BlockSpec((D, 3 * D), lambda p, i: (0, 0)),
            pl.BlockSpec((D, 3 * D), lambda p, i: (0, 0)),
            vec(3 * D), vec(3 * D), vec(D), vec(D), vec(D), vec(D),
        ],
        out_specs=[out_blk(), out_blk(), out_blk(),
                   pl.BlockSpec((NCORES, BLK, DH),
                                lambda p, i: (0, p * i, 0))],
        out_shape=[jax.ShapeDtypeStruct((N, D), jnp.float32)] * 3
        + [jax.ShapeDtypeStruct((NCORES, N, DH), jnp.float32)],
        scratch_shapes=[
            pltpu.VMEM((N, D), jnp.float32),
            pltpu.VMEM((8, D), jnp.float32),
        ],
    )(h, part, m, e, w1t, b1, w2t, b2, wiht, whht, bih, bhh, bng, bnb,
      lng, lnb)


def _tc_maxcombine(partial):
    def body(p_ref, o_ref):
        o_ref[...] = jnp.max(p_ref[...], axis=0)

    return pl.pallas_call(
        body,
        out_shape=jax.ShapeDtypeStruct((G, NREP * D), jnp.float32),
    )(partial)


def kernel(x, edge_index, batch, params):
    src_r = edge_index[0].reshape(NSUB, NWIN, WIN, CH)
    dst_r = edge_index[1].reshape(NSUB, NWIN, WIN, CH)
    zrows = jnp.zeros((RPT, DH), jnp.float32)
    ninf = jnp.full((G, D), -jnp.inf, jnp.float32)

    h = x
    hh = jnp.stack([x[:, :DH], x[:, DH:]])
    m = jnp.ones_like(x)
    e = jnp.zeros_like(x)
    outs = []
    for i in range(L):
        p = params['layers'][i]
        part = _sc_segsum(hh, src_r, dst_r, zrows)
        h, m, e, hh = _tc_layer(
            h, part, m, e, p['w1'].T, p['b1'][None, :],
            p['w2'].T, p['b2'][None, :], p['w_ih'].T, p['w_hh'].T,
            p['b_ih'][None, :], p['b_hh'][None, :],
            p['bn_g'][None, :], p['bn_b'][None, :],
            params['ln_g'][None, :], params['ln_b'][None, :])
        outs.append(h)

    rep = jnp.stack(outs + [m, e])
    partial = _sc_segmax(rep, batch, ninf)
    return _tc_maxcombine(partial)


# speedup vs baseline: 1.3577x; 1.3577x over previous
import functools

import jax
import jax.numpy as jnp
from jax import lax
from jax.experimental import pallas as pl
from jax.experimental.pallas import tpu as pltpu
from jax.experimental.pallas import tpu_sc as plsc

N = 10000
E = 320000
D = 128
G = 256
L = 3

NCORES = 2
NSUB = 16
TILES = NCORES * NSUB
EPT = E // TILES
CH = 125
NCH = EPT // CH
WIN = 16
NWIN = NCH // WIN
RPT = 632
NP = RPT * NSUB

BLK = 1000
NB = N // BLK

RB = 400
RG = N // RB
NREP = 5
NTASK = RG * NREP
TPW = -(-NTASK // TILES)


def _sc_segsum(h, src_r, dst_r, zrows):
    mesh = plsc.VectorSubcoreMesh(core_axis_name="c", subcore_axis_name="s")

    nbuf = 2

    @functools.partial(
        pl.kernel,
        out_type=jax.ShapeDtypeStruct((NCORES, NP, D), jnp.float32),
        mesh=mesh,
        scratch_types=[
            pltpu.VMEM_SHARED((NP, D), jnp.float32),
            pltpu.VMEM((WIN, CH), jnp.int32),
            pltpu.VMEM((WIN, CH), jnp.int32),
            [pltpu.VMEM((CH, D), jnp.float32)] * nbuf,
            [pltpu.SemaphoreType.DMA] * nbuf,
            [pltpu.SemaphoreType.DMA] * nbuf,
        ],
    )
    def k(h_hbm, src_hbm, dst_hbm, z_hbm, out_hbm, acc, sidx, didx, rows,
          gsems, ssems):
        c = lax.axis_index("c")
        s = lax.axis_index("s")
        w = c * NSUB + s
        pltpu.sync_copy(z_hbm, acc.at[pl.ds(s * RPT, RPT)])
        plsc.subcore_barrier()

        def win(g, carry):
            pltpu.sync_copy(src_hbm.at[w, g], sidx)
            pltpu.sync_copy(dst_hbm.at[w, g], didx)

            def body(i, carry2):
                gd = []
                for b in range(nbuf):
                    gd.append(pltpu.async_copy(
                        h_hbm.at[sidx.at[i * nbuf + b]], rows[b], gsems[b]))
                sd = []
                for b in range(nbuf):
                    gd[b].wait()
                    sd.append(pltpu.async_copy(
                        rows[b], acc.at[didx.at[i * nbuf + b]], ssems[b],
                        add=True))
                for b in range(nbuf):
                    sd[b].wait()
                return carry2

            lax.fori_loop(0, WIN // nbuf, body, 0)
            return carry

        lax.fori_loop(0, NWIN, win, 0)
        plsc.subcore_barrier()
        pltpu.sync_copy(acc.at[pl.ds(s * RPT, RPT)],
                        out_hbm.at[c, pl.ds(s * RPT, RPT)])

    return k(h, src_r, dst_r, zrows)


def _sc_segmax(rep, batch, ninf):
    mesh = plsc.VectorSubcoreMesh(core_axis_name="c", subcore_axis_name="s")

    @functools.partial(
        pl.kernel,
        out_type=jax.ShapeDtypeStruct((RG, G, NREP * D), jnp.float32),
        mesh=mesh,
        scratch_types=[
            pltpu.VMEM((RB,), jnp.int32),
            pltpu.VMEM((RB, D), jnp.float32),
            pltpu.VMEM((G, D), jnp.float32),
        ],
        compiler_params=pltpu.CompilerParams(needs_layout_passes=False),
    )
    def k(rep_hbm, batch_hbm, ninf_hbm, out_hbm, ids, buf, acc):
        c = lax.axis_index("c")
        s = lax.axis_index("s")
        w = c * NSUB + s
        lanes = lax.iota(jnp.int32, 16)
        for t in range(TPW):
            task = w + t * TILES

            @pl.when(task < NTASK)
            def _run():
                rg = task // NREP
                a = task % NREP
                pltpu.sync_copy(batch_hbm.at[pl.ds(rg * RB, RB)], ids)
                pltpu.sync_copy(ninf_hbm, acc)
                pltpu.sync_copy(rep_hbm.at[a, pl.ds(rg * RB, RB), :], buf)

                id0 = plsc.load_gather(ids, [jnp.zeros((16,), jnp.int32)])
                init = (id0,) + tuple(
                    jnp.full((16,), -jnp.inf, jnp.float32)
                    for _ in range(D // 16))

                def body(i, carry):
                    pid = carry[0]
                    idv = plsc.load_gather(ids, [jnp.full((16,), i,
                                                          jnp.int32)])
                    new = pid != idv
                    out = [idv]
                    for kk in range(D // 16):
                        rmax = carry[1 + kk]
                        plsc.store_scatter(acc, [pid, lanes + (kk * 16)],
                                           rmax, mask=new)
                        rv = buf[i, pl.ds(kk * 16, 16)]
                        out.append(jnp.where(new, rv, jnp.maximum(rmax, rv)))
                    return tuple(out)

                fin = lax.fori_loop(0, RB, body, init)
                for kk in range(D // 16):
                    plsc.store_scatter(acc, [fin[0], lanes + (kk * 16)],
                                       fin[1 + kk])
                pltpu.sync_copy(acc, out_hbm.at[rg, :, pl.ds(a * D, D)])

    return k(rep, batch, ninf)


def _tc_layer(h, part, m, e, w1t, b1, w2t, b2, wiht, whht, bih, bhh,
              bng, bnb, lng, lnb):
    def body(h_ref, p_ref, m_ref, e_ref, w1_ref, b1_ref, w2_ref, b2_ref,
             wih_ref, whh_ref, bih_ref, bhh_ref, bng_ref, bnb_ref, lng_ref,
             lnb_ref, h_out, m_out, e_out, xg_ref, st_ref):
        p = pl.program_id(0)
        i = pl.program_id(1)

        @pl.when(p == 0)
        def _phase_a():
            gin = h_ref[...] + p_ref[0] + p_ref[1]
            t = jnp.maximum(
                jnp.dot(gin, w1_ref[...], preferred_element_type=jnp.float32)
                + b1_ref[...], 0.0)
            xg = jnp.maximum(
                jnp.dot(t, w2_ref[...], preferred_element_type=jnp.float32)
                + b2_ref[...], 0.0)
            xg_ref[pl.ds(i * BLK, BLK), :] = xg

            @pl.when(i == 0)
            def _init():
                st_ref[...] = jnp.zeros_like(st_ref)

            st_ref[0:1, :] = st_ref[0:1, :] + jnp.sum(xg, axis=0,
                                                      keepdims=True)
            st_ref[1:2, :] = st_ref[1:2, :] + jnp.sum(xg * xg, axis=0,
                                                      keepdims=True)

        @pl.when(p == 1)
        def _phase_b():
            mean = st_ref[0:1, :] * (1.0 / N)
            var = st_ref[1:2, :] * (1.0 / N) - mean * mean
            xn = ((xg_ref[pl.ds(i * BLK, BLK), :] - mean)
                  * lax.rsqrt(var + 1e-5) * bng_ref[...] + bnb_ref[...])
            hprev = h_ref[...]
            gi = jnp.dot(xn, wih_ref[...],
                         preferred_element_type=jnp.float32) + bih_ref[...]
            gh = jnp.dot(hprev, whh_ref[...],
                         preferred_element_type=jnp.float32) + bhh_ref[...]
            r = jax.nn.sigmoid(gi[:, :D] + gh[:, :D])
            z = jax.nn.sigmoid(gi[:, D:2 * D] + gh[:, D:2 * D])
            n = jnp.tanh(gi[:, 2 * D:] + r * gh[:, 2 * D:])
            hn = (1.0 - z) * n + z * hprev
            mu = jnp.mean(hn, axis=1, keepdims=True)
            v2 = jnp.mean((hn - mu) * (hn - mu), axis=1, keepdims=True)
            hl = ((hn - mu) * lax.rsqrt(v2 + 1e-5) * lng_ref[...]
                  + lnb_ref[...])
            h_out[...] = hl
            m_out[...] = m_ref[...] * hl
            e_out[...] = e_ref[...] + hl

    blk = lambda: pl.BlockSpec((BLK, D), lambda p, i: (i, 0))
    out_blk = lambda: pl.BlockSpec((BLK, D), lambda p, i: (p * i, 0))
    vec = lambda n: pl.BlockSpec((1, n), lambda p, i: (0, 0))
    return pl.pallas_call(
        body,
        grid=(2, NB),
        in_specs=[
            blk(),
            pl.BlockSpec((NCORES, BLK, D), lambda p, i: (0, (1 - p) * i, 0)),
            blk(), blk(),
            pl.BlockSpec((D, D), lambda p, i: (0, 0)),
            vec(D),
            pl.BlockSpec((D, D), lambda p, i: (0, 0)),
            vec(D),
            pl.BlockSpec((D, 3 * D), lambda p, i: (0, 0)),
            pl.BlockSpec((D, 3 * D), lambda p, i: (0, 0)),
            vec(3 * D), vec(3 * D), vec(D), vec(D), vec(D), vec(D),
        ],
        out_specs=[out_blk(), out_blk(), out_blk()],
        out_shape=[jax.ShapeDtypeStruct((N, D), jnp.float32)] * 3,
        scratch_shapes=[
            pltpu.VMEM((N, D), jnp.float32),
            pltpu.VMEM((8, D), jnp.float32),
        ],
    )(h, part, m, e, w1t, b1, w2t, b2, wiht, whht, bih, bhh, bng, bnb,
      lng, lnb)


def _tc_maxcombine(partial):
    def body(p_ref, o_ref):
        o_ref[...] = jnp.max(p_ref[...], axis=0)

    return pl.pallas_call(
        body,
        out_shape=jax.ShapeDtypeStruct((G, NREP * D), jnp.float32),
    )(partial)


def kernel(x, edge_index, batch, params):
    src_r = edge_index[0].reshape(TILES, NWIN, WIN, CH)
    dst_r = edge_index[1].reshape(TILES, NWIN, WIN, CH)
    zrows = jnp.zeros((RPT, D), jnp.float32)
    ninf = jnp.full((G, D), -jnp.inf, jnp.float32)

    h = x
    m = jnp.ones_like(x)
    e = jnp.zeros_like(x)
    outs = []
    for i in range(L):
        p = params['layers'][i]
        part = _sc_segsum(h, src_r, dst_r, zrows)
        h, m, e = _tc_layer(
            h, part, m, e, p['w1'].T, p['b1'][None, :],
            p['w2'].T, p['b2'][None, :], p['w_ih'].T, p['w_hh'].T,
            p['b_ih'][None, :], p['b_hh'][None, :],
            p['bn_g'][None, :], p['bn_b'][None, :],
            params['ln_g'][None, :], params['ln_b'][None, :])
        outs.append(h)

    rep = jnp.stack(outs + [m, e])
    partial = _sc_segmax(rep, batch, ninf)
    return _tc_maxcombine(partial)
